# baseline (speedup 1.0000x reference)
import functools

import jax
import jax.numpy as jnp
from jax.experimental import pallas as pl
from jax.experimental.pallas import tpu as pltpu

_E = 64
_TEMP = 0.7
_P_MIN = 0.92
_K = 3
_BAL_W = 0.01
_BLK = 1024


def _gate_body(x_ref, w1t_ref, w2t_ref,
               idx_ref, ts_ref, mask_ref, k_ref, loss_ref, imp_ref):
    h = jnp.tanh(jnp.dot(x_ref[...], w1t_ref[...],
                         preferred_element_type=jnp.float32))
    logits = jnp.dot(h, w2t_ref[...],
                     preferred_element_type=jnp.float32) * (1.0 / _TEMP)
    lt = logits.T

    row = jax.lax.broadcasted_iota(jnp.int32, lt.shape, 0)
    neg_inf = jnp.float32(-jnp.inf)

    m1 = jnp.max(lt, axis=0)
    i1 = jnp.min(jnp.where(lt == m1[None, :], row, _E), axis=0)
    l2 = jnp.where(row == i1[None, :], neg_inf, lt)
    m2 = jnp.max(l2, axis=0)
    i2 = jnp.min(jnp.where(l2 == m2[None, :], row, _E), axis=0)
    l3 = jnp.where(row == i2[None, :], neg_inf, l2)
    m3 = jnp.max(l3, axis=0)
    i3 = jnp.min(jnp.where(l3 == m3[None, :], row, _E), axis=0)

    denom = jnp.sum(jnp.exp(lt - m1[None, :]), axis=0)
    p1 = 1.0 / denom
    p2 = jnp.exp(m2 - m1) / denom
    p3 = jnp.exp(m3 - m1) / denom

    k = jnp.where(p1 >= _P_MIN, 1, jnp.where(p1 + p2 >= _P_MIN, 2, 3))
    k = k.astype(jnp.int32)

    mk2 = (k >= 2).astype(jnp.float32)
    mk3 = (k >= 3).astype(jnp.float32)
    s = p1 + p2 * mk2 + p3 * mk3
    inv = 1.0 / (s + 1e-9)
    ts1 = p1 * inv
    ts2 = p2 * mk2 * inv
    ts3 = p3 * mk3 * inv

    idx_ref[...] = jnp.stack([i1, i2, i3], axis=1)
    ts_ref[...] = jnp.stack([ts1, ts2, ts3], axis=1)
    mask_ref[...] = jnp.stack([jnp.ones_like(mk2), mk2, mk3], axis=1)
    k_ref[...] = k[:, None]

    contrib = (jnp.where(row == i1[None, :], ts1[None, :], 0.0)
               + jnp.where(row == i2[None, :], ts2[None, :], 0.0)
               + jnp.where(row == i3[None, :], ts3[None, :], 0.0))
    part = jnp.sum(contrib, axis=1, keepdims=True)

    @pl.when(pl.program_id(0) == 0)
    def _init():
        imp_ref[...] = part

    @pl.when(pl.program_id(0) != 0)
    def _acc():
        imp_ref[...] = imp_ref[...] + part

    @pl.when(pl.program_id(0) == pl.num_programs(0) - 1)
    def _loss():
        imp = imp_ref[...]
        mean = jnp.sum(imp) * (1.0 / _E)
        var = jnp.sum((imp - mean) ** 2) * (1.0 / _E)
        loss = _BAL_W * var / (mean * mean + 1e-10)
        loss_ref[...] = loss * jnp.ones((1, 1), jnp.float32)


@functools.partial(jax.jit, static_argnames=("interpret",))
def kernel(x, W1, W2, interpret=False):
    n, d = x.shape
    grid = (n // _BLK,)
    out_shapes = (
        jax.ShapeDtypeStruct((n, _K), jnp.int32),
        jax.ShapeDtypeStruct((n, _K), jnp.float32),
        jax.ShapeDtypeStruct((n, _K), jnp.float32),
        jax.ShapeDtypeStruct((n, 1), jnp.int32),
        jax.ShapeDtypeStruct((1, 1), jnp.float32),
        jax.ShapeDtypeStruct((_E, 1), jnp.float32),
    )
    tok_spec = pl.BlockSpec((_BLK, _K), lambda i: (i, 0))
    outs = pl.pallas_call(
        _gate_body,
        grid=grid,
        in_specs=[
            pl.BlockSpec((_BLK, d), lambda i: (i, 0)),
            pl.BlockSpec((d, _E), lambda i: (0, 0)),
            pl.BlockSpec((_E, _E), lambda i: (0, 0)),
        ],
        out_specs=[
            tok_spec,
            tok_spec,
            tok_spec,
            pl.BlockSpec((_BLK, 1), lambda i: (i, 0)),
            pl.BlockSpec((1, 1), lambda i: (0, 0)),
            pl.BlockSpec((_E, 1), lambda i: (0, 0)),
        ],
        out_shape=out_shapes,
        interpret=interpret,
    )(x, W1.T, W2.T)
    top_idx, top_scores, top_mask, k_vec, loss, _ = outs
    return (top_idx, top_scores, top_mask, k_vec.reshape(n),
            loss.reshape(()))
